# baseline (speedup 1.0000x reference)
import functools

import jax
import jax.numpy as jnp
from jax import lax
from jax.experimental import pallas as pl
from jax.experimental.pallas import tpu as pltpu
from jax.experimental.pallas import tpu_sc as plsc

_D = 64
_NI = 16384
_NJ = 50
_NC = 2
_NS = 16
_NW = _NC * _NS
_II = _NI // _NW
_IB = 128
_JC = 2
_NCHUNK = (_II // _IB) * (_NJ // _JC)
_NBUF = 4
_SCALE = 8.0


def _emb_body(xt_hbm, spec_hbm, pretr_hbm, out_hbm, xslab, adj_v, rows_v, obuf,
              spec_v, badc_v, gsem, osem):
    wid = lax.axis_index("s") * _NC + lax.axis_index("c")
    i0w = wid * _II
    pltpu.sync_copy(xt_hbm.at[:, pl.ds(i0w, _II)], xslab)
    pltpu.sync_copy(spec_hbm, spec_v)

    i16 = jnp.arange(16, dtype=jnp.int32)

    def chunk_pos(c):
        ib = c // (_NJ // _JC)
        jc = lax.rem(c, _NJ // _JC)
        return ib * _IB, jc * _JC

    def adj_of(c, slot):
        i_loc, j0 = chunk_pos(c)
        bad = jnp.zeros((16,), jnp.int32)
        for jl in range(_JC):
            for g in range(_IB // 16):
                v = xslab[j0 + jl, pl.ds(i_loc + g * 16, 16)]
                adj_v[slot, jl, pl.ds(g * 16, 16)] = jnp.maximum(v - 5, 0)
                bad = bad | (v < 5).astype(jnp.int32)
        badc_v[slot, pl.ds(0, 16)] = bad

    def fire_gather(slot):
        for jl in range(_JC):
            pltpu.async_copy(
                pretr_hbm.at[adj_v.at[slot, jl]],
                rows_v.at[slot, pl.ds(jl * _IB, _IB)],
                gsem,
            )

    def wait_gather(slot):
        for jl in range(_JC):
            pltpu.make_async_copy(
                pretr_hbm.at[adj_v.at[slot, jl]],
                rows_v.at[slot, pl.ds(jl * _IB, _IB)],
                gsem,
            ).wait()

    def out_slice(c):
        i_loc, j0 = chunk_pos(c)
        return out_hbm.at[pl.ds(j0 * _D, _JC * _D), pl.ds(i0w + i_loc, _IB)]

    def fire_scatter(c, ob):
        pltpu.async_copy(obuf.at[ob], out_slice(c), osem)

    def wait_scatter(c, ob):
        pltpu.make_async_copy(obuf.at[ob], out_slice(c), osem).wait()

    def fix_chunk(c, slot):
        i_loc, j0 = chunk_pos(c)
        slotv = jnp.full((16,), slot, jnp.int32)

        @pl.when(jnp.max(badc_v[slot, pl.ds(0, 16)]) > 0)
        def _():
            for jl in range(_JC):
                for g in range(_IB // 16):
                    v = xslab[j0 + jl, pl.ds(i_loc + g * 16, 16)]
                    mask = v < 5

                    @pl.when(jnp.any(mask))
                    def _():
                        vg = jnp.minimum(v, 4)
                        rowv = i16 + (jl * _IB + g * 16)

                        def fix_col(k, c_):
                            colv = jnp.full((16,), k, jnp.int32)
                            vals = plsc.load_gather(spec_v, [vg, colv], mask=mask)
                            plsc.store_scatter(
                                rows_v, [slotv, rowv, colv], vals, mask=mask
                            )
                            return c_

                        lax.fori_loop(0, _D, fix_col, 0)

    def transpose_scale(slot, ob):
        slotv = jnp.full((16,), slot, jnp.int32)

        def per_k(k, c_):
            colv = jnp.full((16,), k, jnp.int32)
            for jl in range(_JC):
                for g in range(_IB // 16):
                    rowv = i16 + (jl * _IB + g * 16)
                    vals = plsc.load_gather(rows_v, [slotv, rowv, colv])
                    obuf[ob, jl * _D + k, pl.ds(g * 16, 16)] = vals * _SCALE
            return c_

        lax.fori_loop(0, _D, per_k, 0)

    adj_of(0, 0)
    fire_gather(0)
    adj_of(1, 1)
    fire_gather(1)

    def loop_body(c, c_):
        slot = lax.rem(c, _NBUF)
        ob = lax.rem(c, 2)
        wait_gather(slot)

        @pl.when(c + 2 < _NCHUNK)
        def _():
            nslot = lax.rem(c + 2, _NBUF)
            adj_of(c + 2, nslot)
            fire_gather(nslot)

        fix_chunk(c, slot)

        @pl.when(c >= 2)
        def _():
            wait_scatter(c - 2, ob)

        transpose_scale(slot, ob)
        fire_scatter(c, ob)
        return c_

    lax.fori_loop(0, _NCHUNK, loop_body, 0)
    wait_scatter(_NCHUNK - 2, lax.rem(_NCHUNK - 2, 2))
    wait_scatter(_NCHUNK - 1, lax.rem(_NCHUNK - 1, 2))


@jax.jit
def _emb_lookup(xt, special_tokens, pretrained):
    run = functools.partial(
        pl.kernel,
        out_type=jax.ShapeDtypeStruct((_NJ * _D, _NI), jnp.float32),
        mesh=plsc.VectorSubcoreMesh(core_axis_name="c", subcore_axis_name="s"),
        compiler_params=pltpu.CompilerParams(
            use_tc_tiling_on_sc=False, needs_layout_passes=False
        ),
        scratch_types=[
            pltpu.VMEM((_NJ, _II), jnp.int32),
            pltpu.VMEM((_NBUF, _JC, _IB), jnp.int32),
            pltpu.VMEM((_NBUF, _JC * _IB, _D), jnp.float32),
            pltpu.VMEM((2, _JC * _D, _IB), jnp.float32),
            pltpu.VMEM((5, _D), jnp.float32),
            pltpu.VMEM((_NBUF, 16), jnp.int32),
            pltpu.SemaphoreType.DMA,
            pltpu.SemaphoreType.DMA,
        ],
    )(_emb_body)
    return run(xt, special_tokens, pretrained)


def kernel(x, special_tokens, pretrained):
    xt = x.T.astype(jnp.int32)
    outp = _emb_lookup(xt, special_tokens, pretrained)
    return jnp.transpose(outp.reshape(_NJ, _D, _NI), (2, 0, 1))


# speedup vs baseline: 1.1043x; 1.1043x over previous
import functools

import jax
import jax.numpy as jnp
from jax import lax
from jax.experimental import pallas as pl
from jax.experimental.pallas import tpu as pltpu
from jax.experimental.pallas import tpu_sc as plsc

_D = 64
_NI = 16384
_NJ = 50
_NC = 2
_NS = 16
_NW = _NC * _NS
_II = _NI // _NW
_IB = 128
_JC = 2
_NCHUNK = (_II // _IB) * (_NJ // _JC)
_NBUF = 4
_SCALE = 8.0


def _emb_body(xt_hbm, spec_hbm, pretr_hbm, out_hbm, xslab, adj_v, rows_v, obuf,
              spec_v, badc_v, gsem, osem):
    wid = lax.axis_index("s") * _NC + lax.axis_index("c")
    i0w = wid * _II
    pltpu.sync_copy(xt_hbm.at[:, pl.ds(i0w, _II)], xslab)
    pltpu.sync_copy(spec_hbm, spec_v)

    i16 = jnp.arange(16, dtype=jnp.int32)

    def chunk_pos(c):
        ib = c // (_NJ // _JC)
        jc = lax.rem(c, _NJ // _JC)
        return ib * _IB, jc * _JC

    def adj_of(c, slot):
        i_loc, j0 = chunk_pos(c)
        bad = jnp.zeros((16,), jnp.int32)
        for jl in range(_JC):
            for g in range(_IB // 16):
                v = xslab[j0 + jl, pl.ds(i_loc + g * 16, 16)]
                adj_v[slot, jl, pl.ds(g * 16, 16)] = jnp.maximum(v - 5, 0)
                bad = bad | (v < 5).astype(jnp.int32)
        badc_v[slot, pl.ds(0, 16)] = bad

    def fire_gather(slot):
        for jl in range(_JC):
            pltpu.async_copy(
                pretr_hbm.at[adj_v.at[slot, jl]],
                rows_v.at[slot, pl.ds(jl * _IB, _IB)],
                gsem,
            )

    def wait_gather(slot):
        for jl in range(_JC):
            pltpu.make_async_copy(
                pretr_hbm.at[adj_v.at[slot, jl]],
                rows_v.at[slot, pl.ds(jl * _IB, _IB)],
                gsem,
            ).wait()

    def out_slice(c):
        i_loc, j0 = chunk_pos(c)
        icol = (i0w + i_loc) // _IB
        return out_hbm.at[pl.ds(j0, _JC), :, pl.ds(icol, 1), :]

    def fire_scatter(c, ob):
        pltpu.async_copy(obuf.at[ob], out_slice(c), osem)

    def wait_scatter(c, ob):
        pltpu.make_async_copy(obuf.at[ob], out_slice(c), osem).wait()

    def fix_chunk(c, slot):
        i_loc, j0 = chunk_pos(c)
        slotv = jnp.full((16,), slot, jnp.int32)

        @pl.when(jnp.max(badc_v[slot, pl.ds(0, 16)]) > 0)
        def _():
            for jl in range(_JC):
                for g in range(_IB // 16):
                    v = xslab[j0 + jl, pl.ds(i_loc + g * 16, 16)]
                    mask = v < 5

                    @pl.when(jnp.any(mask))
                    def _():
                        vg = jnp.minimum(v, 4)
                        rowv = i16 + (jl * _IB + g * 16)

                        def fix_col(k, c_):
                            colv = jnp.full((16,), k, jnp.int32)
                            vals = plsc.load_gather(spec_v, [vg, colv], mask=mask)
                            plsc.store_scatter(
                                rows_v, [slotv, rowv, colv], vals, mask=mask
                            )
                            return c_

                        lax.fori_loop(0, _D, fix_col, 0)

    def transpose_scale(slot, ob):
        slotv = jnp.full((16,), slot, jnp.int32)
        rowvs = [
            [i16 + (jl * _IB + g * 16) for g in range(_IB // 16)]
            for jl in range(_JC)
        ]

        def per_k(k, c_):
            colv = jnp.full((16,), k, jnp.int32)
            krow = k // 8
            kcol = lax.rem(k, 8) * _IB
            for jl in range(_JC):
                for g in range(_IB // 16):
                    vals = plsc.load_gather(rows_v, [slotv, rowvs[jl][g], colv])
                    obuf[ob, jl, krow, 0, pl.ds(kcol + g * 16, 16)] = vals * _SCALE
            return c_

        lax.fori_loop(0, _D, per_k, 0)

    adj_of(0, 0)
    fire_gather(0)
    adj_of(1, 1)
    fire_gather(1)

    def outer(co, c_):
        for b in range(_NBUF):
            c = co * _NBUF + b
            wait_gather(b)

            @pl.when(c + 2 < _NCHUNK)
            def _():
                nslot = (b + 2) % _NBUF
                adj_of(c + 2, nslot)
                fire_gather(nslot)

            fix_chunk(c, b)

            @pl.when(c >= 2)
            def _():
                wait_scatter(c - 2, b % 2)

            transpose_scale(b, b % 2)
            fire_scatter(c, b % 2)
        return c_

    lax.fori_loop(0, _NCHUNK // _NBUF, outer, 0)
    wait_scatter(_NCHUNK - 2, (_NCHUNK - 2) % 2)
    wait_scatter(_NCHUNK - 1, (_NCHUNK - 1) % 2)


@jax.jit
def _emb_lookup(xt, special_tokens, pretrained):
    run = functools.partial(
        pl.kernel,
        out_type=jax.ShapeDtypeStruct((_NJ, _D // 8, _NI // _IB, 8 * _IB),
                                      jnp.float32),
        mesh=plsc.VectorSubcoreMesh(core_axis_name="c", subcore_axis_name="s"),
        compiler_params=pltpu.CompilerParams(
            use_tc_tiling_on_sc=False, needs_layout_passes=False
        ),
        scratch_types=[
            pltpu.VMEM((_NJ, _II), jnp.int32),
            pltpu.VMEM((_NBUF, _JC, _IB), jnp.int32),
            pltpu.VMEM((_NBUF, _JC * _IB, _D), jnp.float32),
            pltpu.VMEM((2, _JC, _D // 8, 1, 8 * _IB), jnp.float32),
            pltpu.VMEM((5, _D), jnp.float32),
            pltpu.VMEM((_NBUF, 16), jnp.int32),
            pltpu.SemaphoreType.DMA,
            pltpu.SemaphoreType.DMA,
        ],
    )(_emb_body)
    return run(xt, special_tokens, pretrained)


def kernel(x, special_tokens, pretrained):
    xt = jnp.bitwise_and(x.T, jnp.int32(0x7FFFFFFF))
    outp = _emb_lookup(xt, special_tokens, pretrained)
    out5 = outp.reshape(_NJ, _D // 8, _NI // _IB, 8, _IB)
    return jnp.transpose(out5, (2, 4, 0, 1, 3)).reshape(_NI, _NJ, _D)


# speedup vs baseline: 1.8720x; 1.6952x over previous
import functools

import jax
import jax.numpy as jnp
from jax import lax
from jax.experimental import pallas as pl
from jax.experimental.pallas import tpu as pltpu
from jax.experimental.pallas import tpu_sc as plsc

_D = 64
_NI = 16384
_NJ = 50
_NC = 2
_NS = 16
_NW = _NC * _NS
_II = _NI // _NW
_IB = 128
_JC = 2
_NCHUNK = (_II // _IB) * (_NJ // _JC)
_NBUF = 4
_SCALE = 8.0


def _emb_body(xt_hbm, spec_hbm, pretr_hbm, out_hbm, xslab, adj_v, rows_v, obuf,
              spec_v, badc_v, gsem, osem):
    wid = lax.axis_index("s") * _NC + lax.axis_index("c")
    i0w = wid * _II
    pltpu.sync_copy(xt_hbm.at[:, pl.ds(i0w, _II)], xslab)
    pltpu.sync_copy(spec_hbm, spec_v)

    i16 = jnp.arange(16, dtype=jnp.int32)

    def chunk_pos(c):
        ib = c // (_NJ // _JC)
        jc = lax.rem(c, _NJ // _JC)
        return ib * _IB, jc * _JC

    def adj_of(c, slot):
        i_loc, j0 = chunk_pos(c)
        bad = jnp.zeros((16,), jnp.int32)
        for jl in range(_JC):
            for g in range(_IB // 16):
                v = xslab[j0 + jl, pl.ds(i_loc + g * 16, 16)]
                adj_v[slot, jl, pl.ds(g * 16, 16)] = jnp.maximum(v - 5, 0)
                bad = bad | (v < 5).astype(jnp.int32)
        badc_v[slot, pl.ds(0, 16)] = bad

    def fire_gather(slot):
        for jl in range(_JC):
            pltpu.async_copy(
                pretr_hbm.at[adj_v.at[slot, jl]],
                rows_v.at[slot, pl.ds(jl * _IB, _IB)],
                gsem,
            )

    def wait_gather(slot):
        for jl in range(_JC):
            pltpu.make_async_copy(
                pretr_hbm.at[adj_v.at[slot, jl]],
                rows_v.at[slot, pl.ds(jl * _IB, _IB)],
                gsem,
            ).wait()

    def out_slice(c):
        i_loc, j0 = chunk_pos(c)
        icol = (i0w + i_loc) // _IB
        return out_hbm.at[pl.ds(j0, _JC), :, pl.ds(icol, 1), :, :]

    def obuf_slice(ob):
        return obuf.at[ob, :, :, :, :, pl.ds(0, _IB)]

    def fire_scatter(c, ob):
        pltpu.async_copy(obuf_slice(ob), out_slice(c), osem)

    def wait_scatter(c, ob):
        pltpu.make_async_copy(obuf_slice(ob), out_slice(c), osem).wait()

    def fix_chunk(c, slot):
        i_loc, j0 = chunk_pos(c)
        slotv = jnp.full((16,), slot, jnp.int32)

        @pl.when(jnp.max(badc_v[slot, pl.ds(0, 16)]) > 0)
        def _():
            for jl in range(_JC):
                for g in range(_IB // 16):
                    v = xslab[j0 + jl, pl.ds(i_loc + g * 16, 16)]
                    mask = v < 5

                    @pl.when(jnp.any(mask))
                    def _():
                        vg = jnp.minimum(v, 4)
                        rowv = i16 + (jl * _IB + g * 16)

                        def fix_col(k, c_):
                            colv = jnp.full((16,), k, jnp.int32)
                            vals = plsc.load_gather(spec_v, [vg, colv], mask=mask)
                            plsc.store_scatter(
                                rows_v, [slotv, rowv, colv], vals, mask=mask
                            )
                            return c_

                        lax.fori_loop(0, _D, fix_col, 0)

    def transpose_scale(slot, ob):
        obv = jnp.full((16,), ob, jnp.int32)
        zerov = jnp.zeros((16,), jnp.int32)
        krowvs = [(i16 + k4 * 16) // 8 for k4 in range(_D // 16)]
        klovs = [lax.rem(i16 + k4 * 16, 8) for k4 in range(_D // 16)]

        def per_row(r, c_):
            jl = r // _IB
            ilo = lax.rem(r, _IB)
            jlv = jnp.full((16,), jl, jnp.int32)
            ilov = jnp.full((16,), ilo, jnp.int32)
            for k4 in range(_D // 16):
                vals = rows_v[slot, r, pl.ds(k4 * 16, 16)] * _SCALE
                plsc.store_scatter(
                    obuf, [obv, jlv, krowvs[k4], zerov, klovs[k4], ilov], vals
                )
            return c_

        lax.fori_loop(0, _JC * _IB, per_row, 0)

    adj_of(0, 0)
    fire_gather(0)
    adj_of(1, 1)
    fire_gather(1)

    def outer(co, c_):
        for b in range(_NBUF):
            c = co * _NBUF + b
            wait_gather(b)

            @pl.when(c + 2 < _NCHUNK)
            def _():
                nslot = (b + 2) % _NBUF
                adj_of(c + 2, nslot)
                fire_gather(nslot)

            fix_chunk(c, b)

            @pl.when(c >= 2)
            def _():
                wait_scatter(c - 2, b % 2)

            transpose_scale(b, b % 2)
            fire_scatter(c, b % 2)
        return c_

    lax.fori_loop(0, _NCHUNK // _NBUF, outer, 0)
    wait_scatter(_NCHUNK - 2, (_NCHUNK - 2) % 2)
    wait_scatter(_NCHUNK - 1, (_NCHUNK - 1) % 2)


@jax.jit
def _emb_lookup(xt, special_tokens, pretrained):
    run = functools.partial(
        pl.kernel,
        out_type=jax.ShapeDtypeStruct((_NJ, _D // 8, _NI // _IB, 8, _IB),
                                      jnp.float32),
        mesh=plsc.VectorSubcoreMesh(core_axis_name="c", subcore_axis_name="s"),
        compiler_params=pltpu.CompilerParams(
            use_tc_tiling_on_sc=False, needs_layout_passes=False
        ),
        scratch_types=[
            pltpu.VMEM((_NJ, _II), jnp.int32),
            pltpu.VMEM((_NBUF, _JC, _IB), jnp.int32),
            pltpu.VMEM((_NBUF, _JC * _IB, _D), jnp.float32),
            pltpu.VMEM((2, _JC, _D // 8, 1, 8, _IB + 1), jnp.float32),
            pltpu.VMEM((5, _D), jnp.float32),
            pltpu.VMEM((_NBUF, 16), jnp.int32),
            pltpu.SemaphoreType.DMA,
            pltpu.SemaphoreType.DMA,
        ],
    )(_emb_body)
    return run(xt, special_tokens, pretrained)


def kernel(x, special_tokens, pretrained):
    xt = jnp.bitwise_and(x.T, jnp.int32(0x7FFFFFFF))
    out5 = _emb_lookup(xt, special_tokens, pretrained)
    return jnp.transpose(out5, (2, 4, 0, 1, 3)).reshape(_NI, _NJ, _D)
